# baseline (device time: 20446 ns/iter reference)
import jax
import jax.numpy as jnp
from jax import lax
from jax.experimental import pallas as pl
from jax.experimental.pallas import tpu as pltpu

GRID = 8


def kernel(x):
    m, n_local = x.shape
    n_global = 2 * n_local
    bm = m // GRID

    def body(x_ref, out_ref, send_ref, recv_ref, send_sem, recv_sem):
        pi = pl.program_id(0)
        my_x = lax.axis_index("x")
        my_y = lax.axis_index("y")
        nbr = (my_x, 1 - my_y)
        barrier_sem = pltpu.get_barrier_semaphore()

        @pl.when(pi == 0)
        def _():
            pl.semaphore_signal(
                barrier_sem, inc=1, device_id=nbr,
                device_id_type=pl.DeviceIdType.MESH,
            )

        send_ref[pl.ds(pi * bm, bm), :] = jnp.sum(
            x_ref[...], axis=1, keepdims=True, dtype=jnp.float32
        )

        @pl.when(pi == GRID - 1)
        def _():
            pl.semaphore_wait(barrier_sem, 1)
            rdma = pltpu.make_async_remote_copy(
                src_ref=send_ref,
                dst_ref=recv_ref,
                send_sem=send_sem,
                recv_sem=recv_sem,
                device_id=nbr,
                device_id_type=pl.DeviceIdType.MESH,
            )
            rdma.start()
            rdma.wait()
            out_ref[...] = (send_ref[...] + recv_ref[...]) * (1.0 / n_global)

    return pl.pallas_call(
        body,
        grid=(GRID,),
        out_shape=jax.ShapeDtypeStruct((m, 1), jnp.float32),
        in_specs=[
            pl.BlockSpec((bm, n_local), lambda i: (i, 0)),
        ],
        out_specs=pl.BlockSpec((m, 1), lambda i: (0, 0)),
        scratch_shapes=[
            pltpu.VMEM((m, 1), jnp.float32),
            pltpu.VMEM((m, 1), jnp.float32),
            pltpu.SemaphoreType.DMA,
            pltpu.SemaphoreType.DMA,
        ],
        compiler_params=pltpu.CompilerParams(collective_id=0),
    )(x)


# device time: 6686 ns/iter; 3.0580x vs baseline; 3.0580x over previous
import jax
import jax.numpy as jnp
from jax import lax
from jax.experimental import pallas as pl
from jax.experimental.pallas import tpu as pltpu

GRID = 8


def kernel(x):
    m, n_local = x.shape
    n_global = 2 * n_local
    bm = m // GRID

    def body(x_ref, out_ref, send_ref, recv_ref, send_sem, recv_sem):
        pi = pl.program_id(0)
        my_x = lax.axis_index("x")
        my_y = lax.axis_index("y")
        nbr = (my_x, 1 - my_y)
        barrier_sem = pltpu.get_barrier_semaphore()

        @pl.when(pi == 0)
        def _():
            pl.semaphore_signal(
                barrier_sem, inc=1, device_id=nbr,
                device_id_type=pl.DeviceIdType.MESH,
            )

        partial = jnp.sum(x_ref[...], axis=1, keepdims=True, dtype=jnp.float32)
        rows = bm // 128
        send_ref[pl.ds(pi * rows, rows), :] = partial.reshape(rows, 128)

        @pl.when(pi == GRID - 1)
        def _():
            pl.semaphore_wait(barrier_sem, 1)
            rdma = pltpu.make_async_remote_copy(
                src_ref=send_ref,
                dst_ref=recv_ref,
                send_sem=send_sem,
                recv_sem=recv_sem,
                device_id=nbr,
                device_id_type=pl.DeviceIdType.MESH,
            )
            rdma.start()
            rdma.wait()
            out_ref[...] = (send_ref[...] + recv_ref[...]) * (1.0 / n_global)

    packed = pl.pallas_call(
        body,
        grid=(GRID,),
        out_shape=jax.ShapeDtypeStruct((m // 128, 128), jnp.float32),
        in_specs=[
            pl.BlockSpec((bm, n_local), lambda i: (i, 0)),
        ],
        out_specs=pl.BlockSpec((m // 128, 128), lambda i: (0, 0)),
        scratch_shapes=[
            pltpu.VMEM((m // 128, 128), jnp.float32),
            pltpu.VMEM((m // 128, 128), jnp.float32),
            pltpu.SemaphoreType.DMA,
            pltpu.SemaphoreType.DMA,
        ],
        compiler_params=pltpu.CompilerParams(collective_id=0),
    )(x)
    return jnp.reshape(packed, (m, 1))
